# Initial kernel scaffold; baseline (speedup 1.0000x reference)
import jax
import jax.numpy as jnp
from jax.experimental import pallas as pl


def kernel(hidden_states, weight):
    raise NotImplementedError("write your pallas kernel here")


# speedup vs baseline: 1.2598x; 1.2598x over previous
import functools

import jax
import jax.numpy as jnp
from jax.experimental import pallas as pl

TOPK = 8
N_EXPERTS = 64
HIDDEN = 2048
BLOCK_ROWS = 1024


def _gate_kernel(x_ref, w_ref, idx_ref, wgt_ref):
    x = x_ref[...]
    w = w_ref[...]
    logits = jax.lax.dot_general(
        x, w, (((1,), (1,)), ((), ())), preferred_element_type=jnp.float32
    )
    m = jnp.max(logits, axis=-1, keepdims=True)
    e = jnp.exp(logits - m)
    s = jnp.sum(e, axis=-1, keepdims=True)
    scores = e / s

    iota = jax.lax.broadcasted_iota(jnp.int32, scores.shape, 1)
    vals = []
    idxs = []
    work = scores
    for _ in range(TOPK):
        v = jnp.max(work, axis=-1, keepdims=True)
        i = jnp.min(jnp.where(work == v, iota, N_EXPERTS), axis=-1, keepdims=True)
        vals.append(v)
        idxs.append(i)
        work = jnp.where(iota == i, -jnp.inf, work)

    topw = jnp.concatenate(vals, axis=-1)
    topi = jnp.concatenate(idxs, axis=-1)
    denom = jnp.sum(topw, axis=-1, keepdims=True) + 1e-20
    wgt_ref[...] = topw / denom
    idx_ref[...] = topi


@functools.partial(jax.jit, static_argnames=())
def _gate(x, weight):
    n = x.shape[0]
    grid = (n // BLOCK_ROWS,)
    idx, wgt = pl.pallas_call(
        _gate_kernel,
        grid=grid,
        in_specs=[
            pl.BlockSpec((BLOCK_ROWS, HIDDEN), lambda i: (i, 0)),
            pl.BlockSpec((N_EXPERTS, HIDDEN), lambda i: (0, 0)),
        ],
        out_specs=[
            pl.BlockSpec((BLOCK_ROWS, TOPK), lambda i: (i, 0)),
            pl.BlockSpec((BLOCK_ROWS, TOPK), lambda i: (i, 0)),
        ],
        out_shape=[
            jax.ShapeDtypeStruct((n, TOPK), jnp.int32),
            jax.ShapeDtypeStruct((n, TOPK), jnp.float32),
        ],
    )(x, weight)
    return idx, wgt


def kernel(hidden_states, weight):
    b, s, h = hidden_states.shape
    x = hidden_states.reshape(-1, h)
    topk_idx, topk_weight = _gate(x, weight)
    aux_loss = jnp.array(0.0, dtype=jnp.float32)
    return (topk_idx, topk_weight, aux_loss)


# speedup vs baseline: 1.5233x; 1.2091x over previous
import jax
import jax.numpy as jnp
from jax.experimental import pallas as pl

TOPK = 8
N_EXPERTS = 64
HIDDEN = 2048
BLOCK_ROWS = 1024


def _gate_kernel(x_ref, w_ref, idx_ref, wgt_ref):
    x = x_ref[...]
    w = w_ref[...]
    logits = jax.lax.dot_general(
        x, w, (((1,), (0,)), ((), ())), preferred_element_type=jnp.float32
    )

    iota = jax.lax.broadcasted_iota(jnp.int32, logits.shape, 1).astype(jnp.float32)
    vals = []
    idxs = []
    work = logits
    for _ in range(TOPK):
        v = jnp.max(work, axis=-1, keepdims=True)
        i = jnp.min(
            jnp.where(work == v, iota, jnp.float32(N_EXPERTS)),
            axis=-1,
            keepdims=True,
        )
        vals.append(v)
        idxs.append(i)
        work = jnp.where(iota == i, -jnp.inf, work)

    topv = jnp.concatenate(vals, axis=-1)
    topi = jnp.concatenate(idxs, axis=-1)
    e = jnp.exp(topv - vals[0])
    wgt_ref[...] = e / jnp.sum(e, axis=-1, keepdims=True)
    idx_ref[...] = topi.astype(jnp.int32)


def _gate(x, weight_t):
    n = x.shape[0]
    grid = (n // BLOCK_ROWS,)
    idx, wgt = pl.pallas_call(
        _gate_kernel,
        grid=grid,
        in_specs=[
            pl.BlockSpec((BLOCK_ROWS, HIDDEN), lambda i: (i, 0)),
            pl.BlockSpec((HIDDEN, N_EXPERTS), lambda i: (0, 0)),
        ],
        out_specs=[
            pl.BlockSpec((BLOCK_ROWS, TOPK), lambda i: (i, 0)),
            pl.BlockSpec((BLOCK_ROWS, TOPK), lambda i: (i, 0)),
        ],
        out_shape=[
            jax.ShapeDtypeStruct((n, TOPK), jnp.int32),
            jax.ShapeDtypeStruct((n, TOPK), jnp.float32),
        ],
    )(x, weight_t)
    return idx, wgt


def kernel(hidden_states, weight):
    b, s, h = hidden_states.shape
    x = hidden_states.reshape(-1, h)
    topk_idx, topk_weight = _gate(x, weight.T)
    aux_loss = jnp.array(0.0, dtype=jnp.float32)
    return (topk_idx, topk_weight, aux_loss)


# speedup vs baseline: 1.5844x; 1.0401x over previous
import jax
import jax.numpy as jnp
from jax.experimental import pallas as pl

TOPK = 8
N_EXPERTS = 64
HIDDEN = 2048
BLOCK_ROWS = 2048


def _gate_kernel(x_ref, w_ref, idx_ref, wgt_ref):
    x = x_ref[...]
    w = w_ref[...]
    logits = jax.lax.dot_general(
        x, w, (((1,), (0,)), ((), ())), preferred_element_type=jnp.float32
    )

    iota = jax.lax.broadcasted_iota(jnp.int32, logits.shape, 1).astype(jnp.float32)
    vals = []
    idxs = []
    work = logits
    for _ in range(TOPK):
        v = jnp.max(work, axis=-1, keepdims=True)
        i = jnp.min(
            jnp.where(work == v, iota, jnp.float32(N_EXPERTS)),
            axis=-1,
            keepdims=True,
        )
        vals.append(v)
        idxs.append(i)
        work = jnp.where(iota == i, -jnp.inf, work)

    topv = jnp.concatenate(vals, axis=-1)
    topi = jnp.concatenate(idxs, axis=-1)
    e = jnp.exp(topv - vals[0])
    wgt_ref[...] = e / jnp.sum(e, axis=-1, keepdims=True)
    idx_ref[...] = topi.astype(jnp.int32)


def _gate(x, weight_t):
    n = x.shape[0]
    grid = (n // BLOCK_ROWS,)
    idx, wgt = pl.pallas_call(
        _gate_kernel,
        grid=grid,
        in_specs=[
            pl.BlockSpec((BLOCK_ROWS, HIDDEN), lambda i: (i, 0)),
            pl.BlockSpec((HIDDEN, N_EXPERTS), lambda i: (0, 0)),
        ],
        out_specs=[
            pl.BlockSpec((BLOCK_ROWS, TOPK), lambda i: (i, 0)),
            pl.BlockSpec((BLOCK_ROWS, TOPK), lambda i: (i, 0)),
        ],
        out_shape=[
            jax.ShapeDtypeStruct((n, TOPK), jnp.int32),
            jax.ShapeDtypeStruct((n, TOPK), jnp.float32),
        ],
    )(x, weight_t)
    return idx, wgt


def kernel(hidden_states, weight):
    b, s, h = hidden_states.shape
    x = hidden_states.reshape(-1, h)
    topk_idx, topk_weight = _gate(x, weight.T)
    aux_loss = jnp.array(0.0, dtype=jnp.float32)
    return (topk_idx, topk_weight, aux_loss)


# speedup vs baseline: 1.5876x; 1.0021x over previous
import jax
import jax.numpy as jnp
from jax.experimental import pallas as pl
from jax.experimental.pallas import tpu as pltpu

TOPK = 8
N_EXPERTS = 64
HIDDEN = 2048
BLOCK_ROWS = 2048


def _gate_kernel(x_ref, w_ref, idx_ref, wgt_ref):
    logits = jax.lax.dot_general(
        x_ref[...], w_ref[...], (((1,), (0,)), ((), ())),
        preferred_element_type=jnp.float32,
    )

    iota = jax.lax.broadcasted_iota(jnp.int32, logits.shape, 1).astype(jnp.float32)
    vals = []
    idxs = []
    work = logits
    for _ in range(TOPK):
        v = jnp.max(work, axis=-1, keepdims=True)
        i = jnp.min(
            jnp.where(work == v, iota, jnp.float32(N_EXPERTS)),
            axis=-1,
            keepdims=True,
        )
        vals.append(v)
        idxs.append(i)
        work = jnp.where(iota == i, -jnp.inf, work)

    topv = jnp.concatenate(vals, axis=-1)
    topi = jnp.concatenate(idxs, axis=-1)
    e = jnp.exp(topv - vals[0])
    wgt_ref[...] = e / jnp.sum(e, axis=-1, keepdims=True)
    idx_ref[...] = topi.astype(jnp.int32)


def _gate(x, weight_t):
    n = x.shape[0]
    grid = (n // BLOCK_ROWS,)
    idx, wgt = pl.pallas_call(
        _gate_kernel,
        grid=grid,
        in_specs=[
            pl.BlockSpec((BLOCK_ROWS, HIDDEN), lambda i: (i, 0)),
            pl.BlockSpec((HIDDEN, N_EXPERTS), lambda i: (0, 0)),
        ],
        out_specs=[
            pl.BlockSpec((BLOCK_ROWS, TOPK), lambda i: (i, 0)),
            pl.BlockSpec((BLOCK_ROWS, TOPK), lambda i: (i, 0)),
        ],
        out_shape=[
            jax.ShapeDtypeStruct((n, TOPK), jnp.int32),
            jax.ShapeDtypeStruct((n, TOPK), jnp.float32),
        ],
        compiler_params=pltpu.CompilerParams(
            dimension_semantics=("parallel",),
        ),
    )(x, weight_t)
    return idx, wgt


def kernel(hidden_states, weight):
    b, s, h = hidden_states.shape
    x = hidden_states.reshape(-1, h)
    topk_idx, topk_weight = _gate(x, weight.T)
    aux_loss = jnp.array(0.0, dtype=jnp.float32)
    return (topk_idx, topk_weight, aux_loss)
